# baseline (speedup 1.0000x reference)
import jax
import jax.numpy as jnp
from jax.experimental import pallas as pl
from jax.experimental.pallas import tpu as pltpu

_EPS = 1e-5


def _make_block_kernel(H, W, C):
    L = H * W

    def stack_dx(dst_ref, v3):
        z = jnp.zeros((H, 1, C), jnp.float32)
        p = jnp.concatenate([v3[:, 1:, :], z], axis=1)
        m = jnp.concatenate([z, v3[:, :W - 1, :]], axis=1)
        dst_ref[:, 0:C] = p.reshape(L, C).astype(jnp.bfloat16)
        dst_ref[:, C:2 * C] = v3.reshape(L, C).astype(jnp.bfloat16)
        dst_ref[:, 2 * C:3 * C] = m.reshape(L, C).astype(jnp.bfloat16)

    def combine_dy(r):
        zh = jnp.zeros((W, C), jnp.float32)
        dn = jnp.concatenate([zh, r[:L - W, 0:C]], axis=0)
        up = jnp.concatenate([r[W:, 2 * C:3 * C], zh], axis=0)
        return r[:, C:2 * C] + up + dn

    def body(x_ref, wa_ref, wb_ref, s1_ref, b1_ref, s2_ref, b2_ref,
             out_ref, bs_ref):
        B = x_ref.shape[0]
        for b in range(B):
            stack_dx(bs_ref.at[b * L:(b + 1) * L, :], x_ref[b])
        r1 = jnp.dot(bs_ref[...], wa_ref[...],
                     preferred_element_type=jnp.float32)
        ys = []
        for b in range(B):
            rb = r1[b * L:(b + 1) * L, :]
            ys.append(jnp.maximum(
                combine_dy(rb) * s1_ref[...] + b1_ref[...], 0.0))
        for b in range(B):
            stack_dx(bs_ref.at[b * L:(b + 1) * L, :], ys[b].reshape(H, W, C))
        r2 = jnp.dot(bs_ref[...], wb_ref[...],
                     preferred_element_type=jnp.float32)
        for b in range(B):
            rb = r2[b * L:(b + 1) * L, :]
            o = combine_dy(rb) * s2_ref[...] + b2_ref[...] + \
                x_ref[b].reshape(L, C)
            out_ref[b] = jnp.maximum(o, 0.0).reshape(H, W, C)

    return body


def _fold_bn(conv_bias, gamma, beta, mean, var):
    scale = gamma / jnp.sqrt(var + _EPS)
    shift = beta + scale * (conv_bias - mean)
    return scale, shift


def _pack_weights(w):
    cols = [jnp.concatenate([w[ky, 2], w[ky, 1], w[ky, 0]], axis=0)
            for ky in range(3)]
    return jnp.concatenate(cols, axis=1).astype(jnp.bfloat16)


def _basic_block(x_nchw, conv1_w, conv1_b, bn1_gamma, bn1_beta, bn1_mean,
                 bn1_var, conv2_w, conv2_b, bn2_gamma, bn2_beta, bn2_mean,
                 bn2_var, interpret=False):
    N, C, H, W = x_nchw.shape
    L = H * W
    x_nhwc = jnp.transpose(x_nchw, (0, 2, 3, 1))

    s1, b1 = _fold_bn(conv1_b, bn1_gamma, bn1_beta, bn1_mean, bn1_var)
    s2, b2 = _fold_bn(conv2_b, bn2_gamma, bn2_beta, bn2_mean, bn2_var)
    s1 = s1.reshape(1, C).astype(jnp.float32)
    b1 = b1.reshape(1, C).astype(jnp.float32)
    s2 = s2.reshape(1, C).astype(jnp.float32)
    b2 = b2.reshape(1, C).astype(jnp.float32)

    wa = _pack_weights(conv1_w)
    wb = _pack_weights(conv2_w)

    flops = 2 * N * H * W * 9 * (C * C) * 2
    bytes_accessed = 2 * N * C * L * 4 + (wa.size + wb.size) * 2

    B = 2 if N % 2 == 0 else 1
    out_nhwc = pl.pallas_call(
        _make_block_kernel(H, W, C),
        out_shape=jax.ShapeDtypeStruct((N, H, W, C), jnp.float32),
        grid=(N // B,),
        in_specs=[
            pl.BlockSpec((B, H, W, C), lambda n: (n, 0, 0, 0)),
            pl.BlockSpec((3 * C, 3 * C), lambda n: (0, 0)),
            pl.BlockSpec((3 * C, 3 * C), lambda n: (0, 0)),
            pl.BlockSpec((1, C), lambda n: (0, 0)),
            pl.BlockSpec((1, C), lambda n: (0, 0)),
            pl.BlockSpec((1, C), lambda n: (0, 0)),
            pl.BlockSpec((1, C), lambda n: (0, 0)),
        ],
        out_specs=pl.BlockSpec((B, H, W, C), lambda n: (n, 0, 0, 0)),
        scratch_shapes=[
            pltpu.VMEM((B * L, 3 * C), jnp.bfloat16),
        ],
        compiler_params=pltpu.CompilerParams(
            dimension_semantics=("parallel",)),
        cost_estimate=pl.CostEstimate(
            flops=flops, transcendentals=0, bytes_accessed=bytes_accessed),
        interpret=interpret,
    )(x_nhwc, wa, wb, s1, b1, s2, b2)

    return jnp.transpose(out_nhwc, (0, 3, 1, 2))


def kernel(x_nchw, conv1_w, conv1_b, bn1_gamma, bn1_beta, bn1_mean, bn1_var,
           conv2_w, conv2_b, bn2_gamma, bn2_beta, bn2_mean, bn2_var):
    return _basic_block(x_nchw, conv1_w, conv1_b, bn1_gamma, bn1_beta,
                        bn1_mean, bn1_var, conv2_w, conv2_b, bn2_gamma,
                        bn2_beta, bn2_mean, bn2_var)


# speedup vs baseline: 1.0118x; 1.0118x over previous
import jax
import jax.numpy as jnp
from jax.experimental import pallas as pl
from jax.experimental.pallas import tpu as pltpu

_EPS = 1e-5


def _make_block_kernel(H, W, C):
    L = H * W

    def stack_dx(dst_ref, v3):
        z = jnp.zeros((H, 1, C), jnp.float32)
        p = jnp.concatenate([v3[:, 1:, :], z], axis=1)
        m = jnp.concatenate([z, v3[:, :W - 1, :]], axis=1)
        dst_ref[:, 0:C] = p.reshape(L, C).astype(jnp.bfloat16)
        dst_ref[:, C:2 * C] = v3.reshape(L, C).astype(jnp.bfloat16)
        dst_ref[:, 2 * C:3 * C] = m.reshape(L, C).astype(jnp.bfloat16)

    def combine_dy(r):
        zh = jnp.zeros((W, C), jnp.float32)
        dn = jnp.concatenate([zh, r[:L - W, 0:C]], axis=0)
        up = jnp.concatenate([r[W:, 2 * C:3 * C], zh], axis=0)
        return r[:, C:2 * C] + up + dn

    def body(x_ref, wa_ref, wb_ref, s1_ref, b1_ref, s2_ref, b2_ref,
             out_ref, bs_ref):
        for b in range(x_ref.shape[0]):
            x3 = x_ref[b]

            stack_dx(bs_ref, x3)
            r1 = jnp.dot(bs_ref[...], wa_ref[...],
                         preferred_element_type=jnp.float32)
            y = jnp.maximum(combine_dy(r1) * s1_ref[...] + b1_ref[...], 0.0)

            stack_dx(bs_ref, y.reshape(H, W, C))
            r2 = jnp.dot(bs_ref[...], wb_ref[...],
                         preferred_element_type=jnp.float32)
            o = combine_dy(r2) * s2_ref[...] + b2_ref[...] + x3.reshape(L, C)
            out_ref[b] = jnp.maximum(o, 0.0).reshape(H, W, C)

    return body


def _fold_bn(conv_bias, gamma, beta, mean, var):
    scale = gamma / jnp.sqrt(var + _EPS)
    shift = beta + scale * (conv_bias - mean)
    return scale, shift


def _pack_weights(w):
    cols = [jnp.concatenate([w[ky, 2], w[ky, 1], w[ky, 0]], axis=0)
            for ky in range(3)]
    return jnp.concatenate(cols, axis=1).astype(jnp.bfloat16)


def _basic_block(x_nchw, conv1_w, conv1_b, bn1_gamma, bn1_beta, bn1_mean,
                 bn1_var, conv2_w, conv2_b, bn2_gamma, bn2_beta, bn2_mean,
                 bn2_var, interpret=False):
    N, C, H, W = x_nchw.shape
    L = H * W
    x_nhwc = jnp.transpose(x_nchw, (0, 2, 3, 1))

    s1, b1 = _fold_bn(conv1_b, bn1_gamma, bn1_beta, bn1_mean, bn1_var)
    s2, b2 = _fold_bn(conv2_b, bn2_gamma, bn2_beta, bn2_mean, bn2_var)
    s1 = s1.reshape(1, C).astype(jnp.float32)
    b1 = b1.reshape(1, C).astype(jnp.float32)
    s2 = s2.reshape(1, C).astype(jnp.float32)
    b2 = b2.reshape(1, C).astype(jnp.float32)

    wa = _pack_weights(conv1_w)
    wb = _pack_weights(conv2_w)

    flops = 2 * N * H * W * 9 * (C * C) * 2
    bytes_accessed = 2 * N * C * L * 4 + (wa.size + wb.size) * 2

    B = 2 if N % 2 == 0 else 1
    out_nhwc = pl.pallas_call(
        _make_block_kernel(H, W, C),
        out_shape=jax.ShapeDtypeStruct((N, H, W, C), jnp.float32),
        grid=(N // B,),
        in_specs=[
            pl.BlockSpec((B, H, W, C), lambda n: (n, 0, 0, 0)),
            pl.BlockSpec((3 * C, 3 * C), lambda n: (0, 0)),
            pl.BlockSpec((3 * C, 3 * C), lambda n: (0, 0)),
            pl.BlockSpec((1, C), lambda n: (0, 0)),
            pl.BlockSpec((1, C), lambda n: (0, 0)),
            pl.BlockSpec((1, C), lambda n: (0, 0)),
            pl.BlockSpec((1, C), lambda n: (0, 0)),
        ],
        out_specs=pl.BlockSpec((B, H, W, C), lambda n: (n, 0, 0, 0)),
        scratch_shapes=[
            pltpu.VMEM((L, 3 * C), jnp.bfloat16),
        ],
        compiler_params=pltpu.CompilerParams(
            dimension_semantics=("parallel",)),
        cost_estimate=pl.CostEstimate(
            flops=flops, transcendentals=0, bytes_accessed=bytes_accessed),
        interpret=interpret,
    )(x_nhwc, wa, wb, s1, b1, s2, b2)

    return jnp.transpose(out_nhwc, (0, 3, 1, 2))


def kernel(x_nchw, conv1_w, conv1_b, bn1_gamma, bn1_beta, bn1_mean, bn1_var,
           conv2_w, conv2_b, bn2_gamma, bn2_beta, bn2_mean, bn2_var):
    return _basic_block(x_nchw, conv1_w, conv1_b, bn1_gamma, bn1_beta,
                        bn1_mean, bn1_var, conv2_w, conv2_b, bn2_gamma,
                        bn2_beta, bn2_mean, bn2_var)
